# baseline (speedup 1.0000x reference)
import functools

import jax
import jax.numpy as jnp
from jax import lax
from jax.experimental import pallas as pl
from jax.experimental.pallas import tpu as pltpu
from jax.experimental.pallas import tpu_sc as plsc

_N_E_S = 512
_N_E_T = 128
_DS = 1024
_DT = 4
_TN = 1536
_B = 4
_N = 3072


def _vq_tc_body(oh_ref, x_ref, cbs_ref, cbt_ref, idxs_ref, idxt_ref, loss_ref):
    b = pl.program_id(0)
    zs = x_ref[0, 0:_DS, :]
    zt = x_ref[0, _DS:_DS + _DT, :]
    cb = cbs_ref[...]

    m = jax.lax.dot_general(cb, zs, (((1,), (0,)), ((), ())))
    zsq = jnp.sum(zs * zs, axis=0, keepdims=True)
    csq = jnp.sum(cb * cb, axis=1, keepdims=True)
    d = (zsq - 2.0 * m) + csq
    minv = jnp.min(d, axis=0, keepdims=True)
    iota = jax.lax.broadcasted_iota(jnp.int32, (_N_E_S, _TN), 0)
    idx = jnp.min(jnp.where(d == minv, iota, _N_E_S), axis=0)

    ct = (oh_ref[b, 0] * cbt_ref[0] + oh_ref[b, 1] * cbt_ref[1]
          + oh_ref[b, 2] * cbt_ref[2] + oh_ref[b, 3] * cbt_ref[3])
    mt = jax.lax.dot_general(ct, zt, (((1,), (0,)), ((), ())))
    ztsq = jnp.sum(zt * zt, axis=0, keepdims=True)
    ctsq = jnp.sum(ct * ct, axis=1, keepdims=True)
    dt = (ztsq - 2.0 * mt) + ctsq
    minvt = jnp.min(dt, axis=0, keepdims=True)
    iota_t = jax.lax.broadcasted_iota(jnp.int32, (_N_E_T, _TN), 0)
    idxt = jnp.min(jnp.where(dt == minvt, iota_t, _N_E_T), axis=0)

    idxs_ref[0, 0, :] = idx
    idxt_ref[0, 0, :] = idxt
    loss_ref[...] = (jnp.sum(minv) + jnp.sum(minvt)).reshape(1, 1, 1, 1)


def _vq_tc(one_hot, xr, codebook_shared, codebook_task):
    B, C, N = xr.shape
    nt = N // _TN
    grid = (B, nt)
    out_shape = [
        jax.ShapeDtypeStruct((B, 1, N), jnp.int32),
        jax.ShapeDtypeStruct((B, 1, N), jnp.int32),
        jax.ShapeDtypeStruct((B, nt, 1, 1), jnp.float32),
    ]
    in_specs = [
        pl.BlockSpec(memory_space=pltpu.SMEM),
        pl.BlockSpec((1, C, _TN), lambda b, t: (b, 0, t)),
        pl.BlockSpec((_N_E_S, _DS), lambda b, t: (0, 0)),
        pl.BlockSpec((4, _N_E_T, _DT), lambda b, t: (0, 0, 0)),
    ]
    out_specs = [
        pl.BlockSpec((1, 1, _TN), lambda b, t: (b, 0, t)),
        pl.BlockSpec((1, 1, _TN), lambda b, t: (b, 0, t)),
        pl.BlockSpec((1, 1, 1, 1), lambda b, t: (b, t, 0, 0)),
    ]
    return pl.pallas_call(
        _vq_tc_body,
        grid=grid,
        in_specs=in_specs,
        out_specs=out_specs,
        out_shape=out_shape,
        compiler_params=pltpu.CompilerParams(
            dimension_semantics=("parallel", "parallel"),
        ),
    )(one_hot, xr, codebook_shared, codebook_task)


def _sc_gather_body(cbt_hbm, ctt_hbm, idxs_hbm, idxt_hbm, out_hbm,
                    idx_v, table_v, row_v, ttable_v):
    info = plsc.get_sparse_core_info()
    nc = info.num_cores
    wid = lax.axis_index("s") * nc + lax.axis_index("c")
    n_tok = _N // 16
    cpw = _DS // 32

    def tok_body(i, carry):
        sl = pl.ds(i * 16, 16)
        row_v[sl] = plsc.load_gather(table_v, [idx_v[sl]])
        return carry

    def ttok_body(i, carry):
        sl = pl.ds(i * 16, 16)
        row_v[sl] = plsc.load_gather(ttable_v, [idx_v[sl]])
        return carry

    for b in range(_B):
        pltpu.sync_copy(idxs_hbm.at[b], idx_v)

        def chan_body(j, carry, b=b):
            c = wid * cpw + j
            pltpu.sync_copy(cbt_hbm.at[c], table_v)
            lax.fori_loop(0, n_tok, tok_body, 0, unroll=4)
            pltpu.sync_copy(row_v, out_hbm.at[b, c])
            return carry

        lax.fori_loop(0, cpw, chan_body, 0)

    @pl.when(wid < _B * _DT)
    def _():
        b2 = wid // _DT
        dd = wid % _DT
        pltpu.sync_copy(idxt_hbm.at[b2], idx_v)
        pltpu.sync_copy(ctt_hbm.at[b2, dd], ttable_v)
        lax.fori_loop(0, n_tok, ttok_body, 0, unroll=4)
        pltpu.sync_copy(row_v, out_hbm.at[b2, _DS + dd])


def _sc_gather(cbT, ctT, idx_s, idx_t):
    mesh = plsc.VectorSubcoreMesh(core_axis_name="c", subcore_axis_name="s")
    k = functools.partial(
        pl.kernel,
        out_type=jax.ShapeDtypeStruct((_B, _DS + _DT, _N), jnp.float32),
        mesh=mesh,
        compiler_params=pltpu.CompilerParams(needs_layout_passes=False),
        scratch_types=[
            pltpu.VMEM((_N,), jnp.int32),
            pltpu.VMEM((_N_E_S,), jnp.float32),
            pltpu.VMEM((_N,), jnp.float32),
            pltpu.VMEM((_N_E_T,), jnp.float32),
        ],
    )(_sc_gather_body)
    return k(cbT, ctT, idx_s, idx_t)


def kernel(x, one_hot, codebook_shared, codebook_task):
    B, C, D, H, W = x.shape
    N = D * H * W
    xr = x.reshape(B, C, N)

    idxs, idxt, lossp = _vq_tc(one_hot, xr, codebook_shared, codebook_task)
    idx_s = idxs.reshape(B, N)
    idx_t = idxt.reshape(B, N)

    cbT = jnp.transpose(codebook_shared)
    task_idx = jnp.argmax(one_hot, axis=1)
    ctT = jnp.transpose(codebook_task[task_idx], (0, 2, 1))

    out = _sc_gather(cbT, ctT, idx_s, idx_t)

    zq_fold = out.reshape(B, C, D, H, W)
    codebook_loss = 1.25 * jnp.sum(lossp) / (B * N * C)
    return zq_fold, codebook_loss, idx_s, idx_t


# speedup vs baseline: 2.2163x; 2.2163x over previous
import functools

import jax
import jax.numpy as jnp
from jax import lax
from jax.experimental import pallas as pl
from jax.experimental.pallas import tpu as pltpu
from jax.experimental.pallas import tpu_sc as plsc

_N_E_S = 512
_N_E_T = 128
_DS = 1024
_DT = 4
_TN = 1536
_B = 4
_N = 3072


def _vq_tc_body(oh_ref, x_ref, cbs_ref, cbt_ref, idxs_ref, idxt_ref, loss_ref):
    b = pl.program_id(0)
    zs = x_ref[0, 0:_DS, :]
    zt = x_ref[0, _DS:_DS + _DT, :]
    cb = cbs_ref[...]

    m = jax.lax.dot_general(cb, zs, (((1,), (0,)), ((), ())))
    zsq = jnp.sum(zs * zs, axis=0, keepdims=True)
    csq = jnp.sum(cb * cb, axis=1, keepdims=True)
    d = (zsq - 2.0 * m) + csq
    minv = jnp.min(d, axis=0, keepdims=True)
    iota = jax.lax.broadcasted_iota(jnp.int32, (_N_E_S, _TN), 0)
    idx = jnp.min(jnp.where(d == minv, iota, _N_E_S), axis=0)

    ct = (oh_ref[b, 0] * cbt_ref[0] + oh_ref[b, 1] * cbt_ref[1]
          + oh_ref[b, 2] * cbt_ref[2] + oh_ref[b, 3] * cbt_ref[3])
    mt = jax.lax.dot_general(ct, zt, (((1,), (0,)), ((), ())))
    ztsq = jnp.sum(zt * zt, axis=0, keepdims=True)
    ctsq = jnp.sum(ct * ct, axis=1, keepdims=True)
    dt = (ztsq - 2.0 * mt) + ctsq
    minvt = jnp.min(dt, axis=0, keepdims=True)
    iota_t = jax.lax.broadcasted_iota(jnp.int32, (_N_E_T, _TN), 0)
    idxt = jnp.min(jnp.where(dt == minvt, iota_t, _N_E_T), axis=0)

    idxs_ref[0, 0, :] = idx
    idxt_ref[0, 0, :] = idxt
    loss_ref[...] = (jnp.sum(minv) + jnp.sum(minvt)).reshape(1, 1, 1, 1)


def _vq_tc(one_hot, xr, codebook_shared, codebook_task):
    B, C, N = xr.shape
    nt = N // _TN
    grid = (B, nt)
    out_shape = [
        jax.ShapeDtypeStruct((B, 1, N), jnp.int32),
        jax.ShapeDtypeStruct((B, 1, N), jnp.int32),
        jax.ShapeDtypeStruct((B, nt, 1, 1), jnp.float32),
    ]
    in_specs = [
        pl.BlockSpec(memory_space=pltpu.SMEM),
        pl.BlockSpec((1, C, _TN), lambda b, t: (b, 0, t)),
        pl.BlockSpec((_N_E_S, _DS), lambda b, t: (0, 0)),
        pl.BlockSpec((4, _N_E_T, _DT), lambda b, t: (0, 0, 0)),
    ]
    out_specs = [
        pl.BlockSpec((1, 1, _TN), lambda b, t: (b, 0, t)),
        pl.BlockSpec((1, 1, _TN), lambda b, t: (b, 0, t)),
        pl.BlockSpec((1, 1, 1, 1), lambda b, t: (b, t, 0, 0)),
    ]
    return pl.pallas_call(
        _vq_tc_body,
        grid=grid,
        in_specs=in_specs,
        out_specs=out_specs,
        out_shape=out_shape,
        compiler_params=pltpu.CompilerParams(
            dimension_semantics=("parallel", "parallel"),
        ),
    )(one_hot, xr, codebook_shared, codebook_task)


_CPW = 32
_RB = 16


def _sc_gather_body(cbt_hbm, ctt_hbm, idxs_hbm, idxt_hbm, out_hbm,
                    idx_v, tables_v, rows_v, ttable_v):
    info = plsc.get_sparse_core_info()
    nc = info.num_cores
    wid = lax.axis_index("s") * nc + lax.axis_index("c")
    n_tok = _N // 16

    pltpu.sync_copy(cbt_hbm.at[pl.ds(wid * _CPW, _CPW)], tables_v)

    for b in range(_B):
        pltpu.sync_copy(idxs_hbm.at[b], idx_v)
        for half in range(_CPW // _RB):

            @plsc.parallel_loop(0, _RB * n_tok, unroll=8)
            def _(t, half=half):
                j = t // n_tok
                i = t - j * n_tok
                sl = pl.ds(i * 16, 16)
                jv = jnp.full((16,), half * _RB + j, jnp.int32)
                rows_v[j, sl] = plsc.load_gather(tables_v, [jv, idx_v[sl]])

            pltpu.sync_copy(
                rows_v, out_hbm.at[b, pl.ds(wid * _CPW + half * _RB, _RB)])

    @pl.when(wid < _B * _DT)
    def _():
        b2 = wid // _DT
        dd = wid % _DT
        pltpu.sync_copy(idxt_hbm.at[b2], idx_v)
        pltpu.sync_copy(ctt_hbm.at[b2, dd], ttable_v)

        @plsc.parallel_loop(0, n_tok, unroll=8)
        def _(i):
            sl = pl.ds(i * 16, 16)
            rows_v[0, sl] = plsc.load_gather(ttable_v, [idx_v[sl]])

        pltpu.sync_copy(rows_v.at[0], out_hbm.at[b2, _DS + dd])


def _sc_gather(cbT, ctT, idx_s, idx_t):
    mesh = plsc.VectorSubcoreMesh(core_axis_name="c", subcore_axis_name="s")
    k = functools.partial(
        pl.kernel,
        out_type=jax.ShapeDtypeStruct((_B, _DS + _DT, _N), jnp.float32),
        mesh=mesh,
        compiler_params=pltpu.CompilerParams(needs_layout_passes=False),
        scratch_types=[
            pltpu.VMEM((_N,), jnp.int32),
            pltpu.VMEM((_CPW, _N_E_S), jnp.float32),
            pltpu.VMEM((_RB, _N), jnp.float32),
            pltpu.VMEM((_N_E_T,), jnp.float32),
        ],
    )(_sc_gather_body)
    return k(cbT, ctT, idx_s, idx_t)


def kernel(x, one_hot, codebook_shared, codebook_task):
    B, C, D, H, W = x.shape
    N = D * H * W
    xr = x.reshape(B, C, N)

    idxs, idxt, lossp = _vq_tc(one_hot, xr, codebook_shared, codebook_task)
    idx_s = idxs.reshape(B, N)
    idx_t = idxt.reshape(B, N)

    cbT = jnp.transpose(codebook_shared)
    task_idx = jnp.argmax(one_hot, axis=1)
    ctT = jnp.transpose(codebook_task[task_idx], (0, 2, 1))

    out = _sc_gather(cbT, ctT, idx_s, idx_t)

    zq_fold = out.reshape(B, C, D, H, W)
    codebook_loss = 1.25 * jnp.sum(lossp) / (B * N * C)
    return zq_fold, codebook_loss, idx_s, idx_t


# speedup vs baseline: 3.0926x; 1.3954x over previous
import jax
import jax.numpy as jnp
from jax.experimental import pallas as pl
from jax.experimental.pallas import tpu as pltpu

_N_E_S = 512
_N_E_T = 128
_DS = 1024
_DT = 4
_TN = 1536
_TN2 = 1536


def _vq_idx_body(oh_ref, x_ref, cbs_ref, cbt_ref, idxs_ref, idxt_ref, loss_ref):
    b = pl.program_id(0)
    zs = x_ref[0, 0:_DS, :]
    zt = x_ref[0, _DS:_DS + _DT, :]
    cb = cbs_ref[...]

    m = jax.lax.dot_general(cb, zs, (((1,), (0,)), ((), ())))
    zsq = jnp.sum(zs * zs, axis=0, keepdims=True)
    csq = jnp.sum(cb * cb, axis=1, keepdims=True)
    d = (zsq - 2.0 * m) + csq
    minv = jnp.min(d, axis=0, keepdims=True)
    iota = jax.lax.broadcasted_iota(jnp.int32, (_N_E_S, _TN), 0)
    idx = jnp.min(jnp.where(d == minv, iota, _N_E_S), axis=0)

    ct = (oh_ref[b, 0] * cbt_ref[0] + oh_ref[b, 1] * cbt_ref[1]
          + oh_ref[b, 2] * cbt_ref[2] + oh_ref[b, 3] * cbt_ref[3])
    mt = jax.lax.dot_general(ct, zt, (((1,), (0,)), ((), ())))
    ztsq = jnp.sum(zt * zt, axis=0, keepdims=True)
    ctsq = jnp.sum(ct * ct, axis=1, keepdims=True)
    dt = (ztsq - 2.0 * mt) + ctsq
    minvt = jnp.min(dt, axis=0, keepdims=True)
    iota_t = jax.lax.broadcasted_iota(jnp.int32, (_N_E_T, _TN), 0)
    idxt = jnp.min(jnp.where(dt == minvt, iota_t, _N_E_T), axis=0)

    idxs_ref[0, 0, :] = idx
    idxt_ref[0, 0, :] = idxt
    loss_ref[...] = (jnp.sum(minv) + jnp.sum(minvt)).reshape(1, 1, 1, 1)


def _vq_idx(one_hot, xr, codebook_shared, codebook_task):
    B, C, N = xr.shape
    nt = N // _TN
    grid = (B, nt)
    out_shape = [
        jax.ShapeDtypeStruct((B, 1, N), jnp.int32),
        jax.ShapeDtypeStruct((B, 1, N), jnp.int32),
        jax.ShapeDtypeStruct((B, nt, 1, 1), jnp.float32),
    ]
    in_specs = [
        pl.BlockSpec(memory_space=pltpu.SMEM),
        pl.BlockSpec((1, C, _TN), lambda b, t: (b, 0, t)),
        pl.BlockSpec((_N_E_S, _DS), lambda b, t: (0, 0)),
        pl.BlockSpec((4, _N_E_T, _DT), lambda b, t: (0, 0, 0)),
    ]
    out_specs = [
        pl.BlockSpec((1, 1, _TN), lambda b, t: (b, 0, t)),
        pl.BlockSpec((1, 1, _TN), lambda b, t: (b, 0, t)),
        pl.BlockSpec((1, 1, 1, 1), lambda b, t: (b, t, 0, 0)),
    ]
    return pl.pallas_call(
        _vq_idx_body,
        grid=grid,
        in_specs=in_specs,
        out_specs=out_specs,
        out_shape=out_shape,
        compiler_params=pltpu.CompilerParams(
            dimension_semantics=("parallel", "parallel"),
        ),
    )(one_hot, xr, codebook_shared, codebook_task)


def _vq_gather_body(oh_ref, idxs_ref, idxt_ref, cbs_ref, cbt_ref, out_ref):
    b = pl.program_id(0)
    cb = cbs_ref[...]
    idx = idxs_ref[0, 0, :]
    iota = jax.lax.broadcasted_iota(jnp.int32, (_N_E_S, _TN2), 0)
    onehot = (iota == idx[None, :]).astype(jnp.float32)
    zq_s = jax.lax.dot_general(cb, onehot, (((0,), (0,)), ((), ())))

    ct = (oh_ref[b, 0] * cbt_ref[0] + oh_ref[b, 1] * cbt_ref[1]
          + oh_ref[b, 2] * cbt_ref[2] + oh_ref[b, 3] * cbt_ref[3])
    idxt = idxt_ref[0, 0, :]
    iota_t = jax.lax.broadcasted_iota(jnp.int32, (_N_E_T, _TN2), 0)
    onehot_t = (iota_t == idxt[None, :]).astype(jnp.float32)
    zq_t = jax.lax.dot_general(ct, onehot_t, (((0,), (0,)), ((), ())))

    out_ref[0, 0:_DS, :] = zq_s
    out_ref[0, _DS:_DS + _DT, :] = zq_t


def _vq_gather(one_hot, idxs, idxt, codebook_shared, codebook_task, C, N):
    B = idxs.shape[0]
    nt = N // _TN2
    grid = (B, nt)
    in_specs = [
        pl.BlockSpec(memory_space=pltpu.SMEM),
        pl.BlockSpec((1, 1, _TN2), lambda b, t: (b, 0, t)),
        pl.BlockSpec((1, 1, _TN2), lambda b, t: (b, 0, t)),
        pl.BlockSpec((_N_E_S, _DS), lambda b, t: (0, 0)),
        pl.BlockSpec((4, _N_E_T, _DT), lambda b, t: (0, 0, 0)),
    ]
    out_specs = pl.BlockSpec((1, C, _TN2), lambda b, t: (b, 0, t))
    return pl.pallas_call(
        _vq_gather_body,
        grid=grid,
        in_specs=in_specs,
        out_specs=out_specs,
        out_shape=jax.ShapeDtypeStruct((B, C, N), jnp.float32),
        compiler_params=pltpu.CompilerParams(
            dimension_semantics=("parallel", "parallel"),
        ),
    )(one_hot, idxs, idxt, codebook_shared, codebook_task)


def kernel(x, one_hot, codebook_shared, codebook_task):
    B, C, D, H, W = x.shape
    N = D * H * W
    xr = x.reshape(B, C, N)

    idxs, idxt, lossp = _vq_idx(one_hot, xr, codebook_shared, codebook_task)
    out = _vq_gather(one_hot, idxs, idxt, codebook_shared, codebook_task, C, N)

    zq_fold = out.reshape(B, C, D, H, W)
    codebook_loss = 1.25 * jnp.sum(lossp) / (B * N * C)
    return zq_fold, codebook_loss, idxs.reshape(B, N), idxt.reshape(B, N)


# speedup vs baseline: 3.1787x; 1.0278x over previous
import jax
import jax.numpy as jnp
from jax.experimental import pallas as pl
from jax.experimental.pallas import tpu as pltpu

_N_E_S = 512
_N_E_T = 128
_DS = 1024
_DT = 4
_TN = 1536


def _vq_body(oh_ref, x_ref, cbs_ref, cbt_ref, out_ref, idxs_ref, idxt_ref, loss_ref):
    b = pl.program_id(0)
    zs = x_ref[0, 0:_DS, :]
    zt = x_ref[0, _DS:_DS + _DT, :]
    cb = cbs_ref[...]

    m = jax.lax.dot_general(cb, zs, (((1,), (0,)), ((), ())))
    zsq = jnp.sum(zs * zs, axis=0, keepdims=True)
    csq = jnp.sum(cb * cb, axis=1, keepdims=True)
    d = (zsq - 2.0 * m) + csq
    minv = jnp.min(d, axis=0, keepdims=True)
    iota = jax.lax.broadcasted_iota(jnp.int32, (_N_E_S, _TN), 0)
    idx = jnp.min(jnp.where(d == minv, iota, _N_E_S), axis=0)
    onehot = (iota == idx[None, :]).astype(jnp.float32)
    zq_s = jax.lax.dot_general(cb, onehot, (((0,), (0,)), ((), ())))

    ct = (oh_ref[b, 0] * cbt_ref[0] + oh_ref[b, 1] * cbt_ref[1]
          + oh_ref[b, 2] * cbt_ref[2] + oh_ref[b, 3] * cbt_ref[3])
    mt = jax.lax.dot_general(ct, zt, (((1,), (0,)), ((), ())))
    ztsq = jnp.sum(zt * zt, axis=0, keepdims=True)
    ctsq = jnp.sum(ct * ct, axis=1, keepdims=True)
    dt = (ztsq - 2.0 * mt) + ctsq
    minvt = jnp.min(dt, axis=0, keepdims=True)
    iota_t = jax.lax.broadcasted_iota(jnp.int32, (_N_E_T, _TN), 0)
    idxt = jnp.min(jnp.where(dt == minvt, iota_t, _N_E_T), axis=0)
    onehot_t = (iota_t == idxt[None, :]).astype(jnp.float32)
    zq_t = jax.lax.dot_general(ct, onehot_t, (((0,), (0,)), ((), ())))

    out_ref[0, 0:_DS, :] = zq_s
    out_ref[0, _DS:_DS + _DT, :] = zq_t
    idxs_ref[0, 0, :] = idx
    idxt_ref[0, 0, :] = idxt
    loss_ref[...] = (jnp.sum(minv) + jnp.sum(minvt)).reshape(1, 1, 1, 1)


def kernel(x, one_hot, codebook_shared, codebook_task):
    B, C, D, H, W = x.shape
    N = D * H * W
    nt = N // _TN
    xr = x.reshape(B, C, N)

    grid = (B, nt)
    out_shape = [
        jax.ShapeDtypeStruct((B, C, N), jnp.float32),
        jax.ShapeDtypeStruct((B, 1, N), jnp.int32),
        jax.ShapeDtypeStruct((B, 1, N), jnp.int32),
        jax.ShapeDtypeStruct((B, nt, 1, 1), jnp.float32),
    ]
    in_specs = [
        pl.BlockSpec(memory_space=pltpu.SMEM),
        pl.BlockSpec((1, C, _TN), lambda b, t: (b, 0, t)),
        pl.BlockSpec((_N_E_S, _DS), lambda b, t: (0, 0)),
        pl.BlockSpec((4, _N_E_T, _DT), lambda b, t: (0, 0, 0)),
    ]
    out_specs = [
        pl.BlockSpec((1, C, _TN), lambda b, t: (b, 0, t)),
        pl.BlockSpec((1, 1, _TN), lambda b, t: (b, 0, t)),
        pl.BlockSpec((1, 1, _TN), lambda b, t: (b, 0, t)),
        pl.BlockSpec((1, 1, 1, 1), lambda b, t: (b, t, 0, 0)),
    ]
    out, idxs, idxt, lossp = pl.pallas_call(
        _vq_body,
        grid=grid,
        in_specs=in_specs,
        out_specs=out_specs,
        out_shape=out_shape,
        compiler_params=pltpu.CompilerParams(
            dimension_semantics=("parallel", "parallel"),
        ),
    )(one_hot, xr, codebook_shared, codebook_task)

    zq_fold = out.reshape(B, C, D, H, W)
    codebook_loss = 1.25 * jnp.sum(lossp) / (B * N * C)
    return zq_fold, codebook_loss, idxs.reshape(B, N), idxt.reshape(B, N)
